# baseline (speedup 1.0000x reference)
import functools

import jax
import jax.numpy as jnp
from jax.experimental import pallas as pl
from jax.experimental.pallas import tpu as pltpu
from jax.experimental.pallas import tpu_sc as plsc

SEQ = 512
DM = 1024
NE = 6
TOPK = 4
TILE = 512
NC = 2
NS = 16
LANES = 16


def _expert_dims(pl_e):
    step = pl_e // 2
    pn = int((SEQ - pl_e) / step + 1)
    return step, pn


def _gate_sc_body(tpw, t_total, l_hbm, g_hbm, lv, gv):
    wid = jax.lax.axis_index("s") * NC + jax.lax.axis_index("c")
    base = wid * tpw
    pltpu.sync_copy(l_hbm, lv)
    zero = jnp.zeros((LANES,), jnp.float32)
    for g in range(tpw // LANES):
        lvec = [lv[pl.ds(e * t_total + base + g * LANES, LANES)]
                for e in range(NE)]
        rank = []
        for e in range(NE):
            r = jnp.zeros((LANES,), jnp.float32)
            for j in range(NE):
                if j == e:
                    continue
                beats = (lvec[j] > lvec[e]) if j > e else (lvec[j] >= lvec[e])
                r = r + jnp.where(beats, 1.0, 0.0)
            rank.append(r)
        mx = lvec[0]
        for e in range(1, NE):
            mx = jnp.maximum(mx, lvec[e])
        pv = [jnp.where(rank[e] < TOPK, jnp.exp(lvec[e] - mx), 0.0)
              for e in range(NE)]
        inv = 1.0 / (pv[0] + pv[1] + pv[2] + pv[3] + pv[4] + pv[5])
        for e in range(NE):
            gv[pl.ds(e * tpw + g * LANES, LANES)] = pv[e] * inv
        gv[pl.ds(NE * tpw + g * LANES, LANES)] = zero
        gv[pl.ds((NE + 1) * tpw + g * LANES, LANES)] = zero
    pltpu.sync_copy(gv, g_hbm.at[pl.ds(wid * 8 * tpw, 8 * tpw)])


def _logits_body(x_ref, gw_ref, gb_ref, l_ref):
    lt = jax.lax.dot_general(gw_ref[...], x_ref[...],
                             dimension_numbers=(((0,), (1,)), ((), ())),
                             preferred_element_type=jnp.float32)
    l_ref[0:NE, :] = lt + gb_ref[...]
    l_ref[NE:8, :] = jnp.zeros((8 - NE, l_ref.shape[1]), jnp.float32)


def _compose_body(dims, *refs):
    wf_refs = refs[:NE]
    w1_refs = refs[NE:2 * NE]
    bf_ref, b1_ref, a_ref, m_ref, w1s_ref = refs[2 * NE:]

    rows = jax.lax.broadcasted_iota(jnp.int32, (SEQ + 1, DM), 0)
    cols = jax.lax.broadcasted_iota(jnp.int32, (SEQ + 1, DM), 1)
    for e, (pl_e, step, pn, dm) in enumerate(dims):
        if pl_e >= 24:
            m_ref[...] = jnp.zeros((SEQ + 1, DM), jnp.bfloat16)
            wf = wf_refs[e][...]
            for n in range(pn):
                m_ref[n * step:n * step + pl_e, n * dm:(n + 1) * dm] = wf
            m_ref[SEQ:SEQ + 1, :] = bf_ref[e:e + 1, :].astype(jnp.bfloat16)
            mblk = m_ref[...]
        else:
            n = cols // dm
            p = jnp.where((cols < pn * dm) & (rows < SEQ),
                          rows - n * step, -1)
            blk = jnp.where(rows == SEQ, bf_ref[e:e + 1, :], 0.0)
            for pp in range(pl_e):
                blk = blk + jnp.where(p == pp,
                                      wf_refs[e][pp:pp + 1,
                                                 :].astype(jnp.float32),
                                      0.0)
            mblk = blk.astype(jnp.bfloat16)
        nk = pn * dm
        w1s_ref[0:nk, :] = w1_refs[e][...].astype(jnp.bfloat16)
        if nk < DM:
            w1s_ref[nk:DM, :] = jnp.zeros((DM - nk, DM), jnp.bfloat16)
        a_ref[:, e * DM:(e + 1) * DM] = jnp.dot(
            mblk, w1s_ref[...],
            preferred_element_type=jnp.float32).astype(jnp.bfloat16)
        a_ref[SEQ:SEQ + 1, e * DM:(e + 1) * DM] = (
            a_ref[SEQ:SEQ + 1, e * DM:(e + 1) * DM]
            + b1_ref[e:e + 1, :].astype(jnp.bfloat16))


def _main_body(x_ref, g_ref, a_ref, o_ref):
    xb16 = x_ref[...].astype(jnp.bfloat16)
    gcol = jnp.transpose(g_ref[...])
    y = jnp.dot(xb16, a_ref[0:SEQ, :],
                preferred_element_type=jnp.float32)
    y = y + a_ref[SEQ:SEQ + 1, :].astype(jnp.float32)
    acc = jnp.zeros((TILE, DM), jnp.float32)
    for e in range(NE):
        acc = acc + gcol[:, e:e + 1] * y[:, e * DM:(e + 1) * DM]
    o_ref[...] = acc


def kernel(x, ff_W, ff_b, ff1_W, ff1_b, gate_W, gate_b):
    B, V, S = x.shape
    T = B * V
    dims = []
    for e in range(NE):
        pl_e, dm = ff_W[e].shape
        step, pn = _expert_dims(pl_e)
        dims.append((pl_e, step, pn, dm))
    dims = tuple(dims)

    xf = x.reshape(T, S)
    bf_tiled = jnp.stack([
        jnp.pad(jnp.tile(ff_b[e], dims[e][2]),
                (0, DM - dims[e][2] * dims[e][3]))
        for e in range(NE)])
    b1_stack = jnp.stack(ff1_b)
    wf_args = []
    for e, (pl_e, step, pn, dm) in enumerate(dims):
        if pl_e >= 24:
            wf_args.append(ff_W[e].astype(jnp.bfloat16))
        else:
            wf_args.append(jnp.pad(jnp.tile(ff_W[e], (1, pn)),
                                   ((0, 0), (0, DM - pn * dm))))

    full = lambda shape: pl.BlockSpec(shape, lambda: (0,) * len(shape))

    logits_t = pl.pallas_call(
        _logits_body,
        out_shape=jax.ShapeDtypeStruct((8, T), jnp.float32),
        in_specs=[full((T, S)), full((S, NE)), full((NE, 1))],
        out_specs=full((8, T)),
    )(xf, gate_W, gate_b[:, None])

    a_mat = pl.pallas_call(
        functools.partial(_compose_body, dims),
        out_shape=jax.ShapeDtypeStruct((SEQ + 1, NE * DM), jnp.bfloat16),
        in_specs=[full(w.shape) for w in wf_args]
        + [full(w.shape) for w in ff1_W]
        + [full((NE, DM)), full((NE, DM))],
        out_specs=full((SEQ + 1, NE * DM)),
        scratch_shapes=[pltpu.VMEM((SEQ + 1, DM), jnp.bfloat16),
                        pltpu.VMEM((DM, DM), jnp.bfloat16)],
    )(*wf_args, *ff1_W, bf_tiled, b1_stack)

    tpw = T // (NC * NS)
    gates_w = pl.kernel(
        functools.partial(_gate_sc_body, tpw, T),
        out_type=jax.ShapeDtypeStruct((8 * T,), jnp.float32),
        mesh=plsc.VectorSubcoreMesh(core_axis_name="c", subcore_axis_name="s",
                                    num_cores=NC, num_subcores=NS),
        scratch_types=[pltpu.VMEM((8 * T,), jnp.float32),
                       pltpu.VMEM((8 * tpw,), jnp.float32)],
    )(logits_t.reshape(8 * T))
    gates_t = gates_w.reshape(NC * NS, 8, tpw).transpose(1, 0, 2)

    grid = (T // TILE,)
    out = pl.pallas_call(
        _main_body,
        grid=grid,
        out_shape=jax.ShapeDtypeStruct((T, DM), jnp.float32),
        in_specs=[
            pl.BlockSpec((TILE, S), lambda i: (i, 0)),
            pl.BlockSpec((8, TILE), lambda i: (0, i)),
            pl.BlockSpec((SEQ + 1, NE * DM), lambda i: (0, 0)),
        ],
        out_specs=pl.BlockSpec((TILE, DM), lambda i: (i, 0)),
    )(xf, gates_t.reshape(8, T), a_mat)

    return out.reshape(B, V, DM)


# speedup vs baseline: 1.0610x; 1.0610x over previous
import functools

import jax
import jax.numpy as jnp
from jax.experimental import pallas as pl
from jax.experimental.pallas import tpu as pltpu
from jax.experimental.pallas import tpu_sc as plsc

SEQ = 512
DM = 1024
NE = 6
TOPK = 4
TILE = 256
NC = 2
NS = 16
LANES = 16


def _expert_dims(pl_e):
    step = pl_e // 2
    pn = int((SEQ - pl_e) / step + 1)
    return step, pn


def _gate_sc_body(tpw, t_total, l_hbm, g_hbm, lv, gv):
    wid = jax.lax.axis_index("s") * NC + jax.lax.axis_index("c")
    base = wid * tpw
    pltpu.sync_copy(l_hbm, lv)
    zero = jnp.zeros((LANES,), jnp.float32)
    for g in range(tpw // LANES):
        lvec = [lv[pl.ds(e * t_total + base + g * LANES, LANES)]
                for e in range(NE)]
        rank = []
        for e in range(NE):
            r = jnp.zeros((LANES,), jnp.float32)
            for j in range(NE):
                if j == e:
                    continue
                beats = (lvec[j] > lvec[e]) if j > e else (lvec[j] >= lvec[e])
                r = r + jnp.where(beats, 1.0, 0.0)
            rank.append(r)
        mx = lvec[0]
        for e in range(1, NE):
            mx = jnp.maximum(mx, lvec[e])
        pv = [jnp.where(rank[e] < TOPK, jnp.exp(lvec[e] - mx), 0.0)
              for e in range(NE)]
        inv = 1.0 / (pv[0] + pv[1] + pv[2] + pv[3] + pv[4] + pv[5])
        for e in range(NE):
            gv[pl.ds(e * tpw + g * LANES, LANES)] = pv[e] * inv
        gv[pl.ds(NE * tpw + g * LANES, LANES)] = zero
        gv[pl.ds((NE + 1) * tpw + g * LANES, LANES)] = zero
    pltpu.sync_copy(gv, g_hbm.at[pl.ds(wid * 8 * tpw, 8 * tpw)])


def _logits_body(x_ref, gw_ref, gb_ref, l_ref):
    lt = jax.lax.dot_general(gw_ref[...], x_ref[...],
                             dimension_numbers=(((0,), (1,)), ((), ())),
                             preferred_element_type=jnp.float32)
    l_ref[0:NE, :] = lt + gb_ref[...]
    l_ref[NE:8, :] = jnp.zeros((8 - NE, l_ref.shape[1]), jnp.float32)


def _compose_into(dims, wf_refs, w1_refs, bf_ref, b1_ref, a_ref, m_ref,
                  w1s_ref):
    rows = jax.lax.broadcasted_iota(jnp.int32, (SEQ + 1, DM), 0)
    cols = jax.lax.broadcasted_iota(jnp.int32, (SEQ + 1, DM), 1)
    for e, (pl_e, step, pn, dm) in enumerate(dims):
        if pl_e >= 24:
            m_ref[...] = jnp.zeros((SEQ + 1, DM), jnp.bfloat16)
            wf = wf_refs[e][...]
            for n in range(pn):
                m_ref[n * step:n * step + pl_e, n * dm:(n + 1) * dm] = wf
            m_ref[SEQ:SEQ + 1, :] = bf_ref[e:e + 1, :].astype(jnp.bfloat16)
            mblk = m_ref[...]
        else:
            n = cols // dm
            p = jnp.where((cols < pn * dm) & (rows < SEQ),
                          rows - n * step, -1)
            blk = jnp.where(rows == SEQ, bf_ref[e:e + 1, :], 0.0)
            for pp in range(pl_e):
                blk = blk + jnp.where(p == pp,
                                      wf_refs[e][pp:pp + 1,
                                                 :].astype(jnp.float32),
                                      0.0)
            mblk = blk.astype(jnp.bfloat16)
        nk = pn * dm
        w1s_ref[0:nk, :] = w1_refs[e][...].astype(jnp.bfloat16)
        if nk < DM:
            w1s_ref[nk:DM, :] = jnp.zeros((DM - nk, DM), jnp.bfloat16)
        a_ref[:, e * DM:(e + 1) * DM] = jnp.dot(
            mblk, w1s_ref[...],
            preferred_element_type=jnp.float32).astype(jnp.bfloat16)
        a_ref[SEQ:SEQ + 1, e * DM:(e + 1) * DM] = (
            a_ref[SEQ:SEQ + 1, e * DM:(e + 1) * DM]
            + b1_ref[e:e + 1, :].astype(jnp.bfloat16))


def _fused_body(dims, *refs):
    x_ref, g_ref = refs[:2]
    wf_refs = refs[2:2 + NE]
    w1_refs = refs[2 + NE:2 + 2 * NE]
    bf_ref, b1_ref, o_ref, a_ref, m_ref, w1s_ref = refs[2 + 2 * NE:]
    i = pl.program_id(0)

    @pl.when(i == 0)
    def _compose():
        _compose_into(dims, wf_refs, w1_refs, bf_ref, b1_ref, a_ref,
                      m_ref, w1s_ref)

    @pl.when(i > 0)
    def _tile():
        xb16 = x_ref[...].astype(jnp.bfloat16)
        gcol = jnp.transpose(g_ref[...])
        y = jnp.dot(xb16, a_ref[0:SEQ, :],
                    preferred_element_type=jnp.float32)
        y = y + a_ref[SEQ:SEQ + 1, :].astype(jnp.float32)
        acc = jnp.zeros((TILE, DM), jnp.float32)
        for e in range(NE):
            acc = acc + gcol[:, e:e + 1] * y[:, e * DM:(e + 1) * DM]
        o_ref[...] = acc


def kernel(x, ff_W, ff_b, ff1_W, ff1_b, gate_W, gate_b):
    B, V, S = x.shape
    T = B * V
    dims = []
    for e in range(NE):
        pl_e, dm = ff_W[e].shape
        step, pn = _expert_dims(pl_e)
        dims.append((pl_e, step, pn, dm))
    dims = tuple(dims)

    xf = x.reshape(T, S)
    bf_tiled = jnp.stack([
        jnp.pad(jnp.tile(ff_b[e], dims[e][2]),
                (0, DM - dims[e][2] * dims[e][3]))
        for e in range(NE)])
    b1_stack = jnp.stack(ff1_b)
    wf_args = []
    for e, (pl_e, step, pn, dm) in enumerate(dims):
        if pl_e >= 24:
            wf_args.append(ff_W[e].astype(jnp.bfloat16))
        else:
            wf_args.append(jnp.pad(jnp.tile(ff_W[e], (1, pn)),
                                   ((0, 0), (0, DM - pn * dm))))

    full = lambda shape: pl.BlockSpec(shape, lambda: (0,) * len(shape))

    logits_t = pl.pallas_call(
        _logits_body,
        out_shape=jax.ShapeDtypeStruct((8, T), jnp.float32),
        in_specs=[full((T, S)), full((S, NE)), full((NE, 1))],
        out_specs=full((8, T)),
    )(xf, gate_W, gate_b[:, None])

    tpw = T // (NC * NS)
    gates_w = pl.kernel(
        functools.partial(_gate_sc_body, tpw, T),
        out_type=jax.ShapeDtypeStruct((8 * T,), jnp.float32),
        mesh=plsc.VectorSubcoreMesh(core_axis_name="c", subcore_axis_name="s",
                                    num_cores=NC, num_subcores=NS),
        scratch_types=[pltpu.VMEM((8 * T,), jnp.float32),
                       pltpu.VMEM((8 * tpw,), jnp.float32)],
    )(logits_t.reshape(8 * T))
    gates_t = gates_w.reshape(NC * NS, 8, tpw).transpose(1, 0, 2)

    grid = (T // TILE + 1,)
    shift = lambda i: jnp.maximum(i - 1, 0)
    full_g = lambda shape: pl.BlockSpec(shape,
                                        lambda i: (0,) * len(shape))
    out = pl.pallas_call(
        functools.partial(_fused_body, dims),
        grid=grid,
        out_shape=jax.ShapeDtypeStruct((T, DM), jnp.float32),
        in_specs=[
            pl.BlockSpec((TILE, S), lambda i: (shift(i), 0)),
            pl.BlockSpec((8, TILE), lambda i: (0, shift(i))),
        ]
        + [full_g(w.shape) for w in wf_args]
        + [full_g(w.shape) for w in ff1_W]
        + [full_g((NE, DM)), full_g((NE, DM))],
        out_specs=pl.BlockSpec((TILE, DM), lambda i: (shift(i), 0)),
        scratch_shapes=[pltpu.VMEM((SEQ + 1, NE * DM), jnp.bfloat16),
                        pltpu.VMEM((SEQ + 1, DM), jnp.bfloat16),
                        pltpu.VMEM((DM, DM), jnp.bfloat16)],
    )(xf, gates_t.reshape(8, T), *wf_args, *ff1_W, bf_tiled, b1_stack)

    return out.reshape(B, V, DM)


# speedup vs baseline: 1.0815x; 1.0193x over previous
import functools

import jax
import jax.numpy as jnp
from jax.experimental import pallas as pl
from jax.experimental.pallas import tpu as pltpu
from jax.experimental.pallas import tpu_sc as plsc

SEQ = 512
DM = 1024
NE = 6
TOPK = 4
TILE = 512
NC = 2
NS = 16
LANES = 16


def _expert_dims(pl_e):
    step = pl_e // 2
    pn = int((SEQ - pl_e) / step + 1)
    return step, pn


def _gate_sc_body(tpw, t_total, l_hbm, g_hbm, lv, gv):
    wid = jax.lax.axis_index("s") * NC + jax.lax.axis_index("c")
    base = wid * tpw
    pltpu.sync_copy(l_hbm, lv)
    zero = jnp.zeros((LANES,), jnp.float32)
    for g in range(tpw // LANES):
        lvec = [lv[pl.ds(e * t_total + base + g * LANES, LANES)]
                for e in range(NE)]
        rank = []
        for e in range(NE):
            r = jnp.zeros((LANES,), jnp.float32)
            for j in range(NE):
                if j == e:
                    continue
                beats = (lvec[j] > lvec[e]) if j > e else (lvec[j] >= lvec[e])
                r = r + jnp.where(beats, 1.0, 0.0)
            rank.append(r)
        mx = lvec[0]
        for e in range(1, NE):
            mx = jnp.maximum(mx, lvec[e])
        pv = [jnp.where(rank[e] < TOPK, jnp.exp(lvec[e] - mx), 0.0)
              for e in range(NE)]
        inv = 1.0 / (pv[0] + pv[1] + pv[2] + pv[3] + pv[4] + pv[5])
        for e in range(NE):
            gv[pl.ds(e * tpw + g * LANES, LANES)] = pv[e] * inv
        gv[pl.ds(NE * tpw + g * LANES, LANES)] = zero
        gv[pl.ds((NE + 1) * tpw + g * LANES, LANES)] = zero
    pltpu.sync_copy(gv, g_hbm.at[pl.ds(wid * 8 * tpw, 8 * tpw)])


def _logits_body(x_ref, gw_ref, gb_ref, l_ref):
    lt = jax.lax.dot_general(gw_ref[...], x_ref[...],
                             dimension_numbers=(((0,), (1,)), ((), ())),
                             preferred_element_type=jnp.float32)
    l_ref[0:NE, :] = lt + gb_ref[...]
    l_ref[NE:8, :] = jnp.zeros((8 - NE, l_ref.shape[1]), jnp.float32)


def _compose_into(dims, wf_refs, w1_refs, bf_ref, b1_ref, a_ref, m_ref,
                  w1s_ref):
    rows = jax.lax.broadcasted_iota(jnp.int32, (SEQ + 1, DM), 0)
    cols = jax.lax.broadcasted_iota(jnp.int32, (SEQ + 1, DM), 1)
    for e, (pl_e, step, pn, dm) in enumerate(dims):
        if pl_e >= 24:
            m_ref[...] = jnp.zeros((SEQ + 1, DM), jnp.bfloat16)
            wf = wf_refs[e][...]
            for n in range(pn):
                m_ref[n * step:n * step + pl_e, n * dm:(n + 1) * dm] = wf
            m_ref[SEQ:SEQ + 1, :] = bf_ref[e:e + 1, :].astype(jnp.bfloat16)
            mblk = m_ref[...]
        else:
            n = cols // dm
            p = jnp.where((cols < pn * dm) & (rows < SEQ),
                          rows - n * step, -1)
            blk = jnp.where(rows == SEQ, bf_ref[e:e + 1, :], 0.0)
            for pp in range(pl_e):
                blk = blk + jnp.where(p == pp,
                                      wf_refs[e][pp:pp + 1,
                                                 :].astype(jnp.float32),
                                      0.0)
            mblk = blk.astype(jnp.bfloat16)
        nk = pn * dm
        w1s_ref[0:nk, :] = w1_refs[e][...].astype(jnp.bfloat16)
        if nk < DM:
            w1s_ref[nk:DM, :] = jnp.zeros((DM - nk, DM), jnp.bfloat16)
        a_ref[:, e * DM:(e + 1) * DM] = jnp.dot(
            mblk, w1s_ref[...],
            preferred_element_type=jnp.float32).astype(jnp.bfloat16)
        a_ref[SEQ:SEQ + 1, e * DM:(e + 1) * DM] = (
            a_ref[SEQ:SEQ + 1, e * DM:(e + 1) * DM]
            + b1_ref[e:e + 1, :].astype(jnp.bfloat16))


def _fused_body(dims, *refs):
    x_ref, g_ref = refs[:2]
    wf_refs = refs[2:2 + NE]
    w1_refs = refs[2 + NE:2 + 2 * NE]
    bf_ref, b1_ref, o_ref, a_ref, m_ref, w1s_ref = refs[2 + 2 * NE:]
    i = pl.program_id(0)

    @pl.when(i == 0)
    def _compose():
        _compose_into(dims, wf_refs, w1_refs, bf_ref, b1_ref, a_ref,
                      m_ref, w1s_ref)

    @pl.when(i > 0)
    def _tile():
        xb16 = x_ref[...].astype(jnp.bfloat16)
        gcol = jnp.transpose(g_ref[...])
        y = jnp.dot(xb16, a_ref[0:SEQ, :],
                    preferred_element_type=jnp.float32)
        y = y + a_ref[SEQ:SEQ + 1, :].astype(jnp.float32)
        acc = jnp.zeros((TILE, DM), jnp.float32)
        for e in range(NE):
            acc = acc + gcol[:, e:e + 1] * y[:, e * DM:(e + 1) * DM]
        o_ref[...] = acc


def kernel(x, ff_W, ff_b, ff1_W, ff1_b, gate_W, gate_b):
    B, V, S = x.shape
    T = B * V
    dims = []
    for e in range(NE):
        pl_e, dm = ff_W[e].shape
        step, pn = _expert_dims(pl_e)
        dims.append((pl_e, step, pn, dm))
    dims = tuple(dims)

    xf = x.reshape(T, S)
    bf_tiled = jnp.stack([
        jnp.pad(jnp.tile(ff_b[e], dims[e][2]),
                (0, DM - dims[e][2] * dims[e][3]))
        for e in range(NE)])
    b1_stack = jnp.stack(ff1_b)
    wf_args = []
    for e, (pl_e, step, pn, dm) in enumerate(dims):
        if pl_e >= 24:
            wf_args.append(ff_W[e].astype(jnp.bfloat16))
        else:
            wf_args.append(jnp.pad(jnp.tile(ff_W[e], (1, pn)),
                                   ((0, 0), (0, DM - pn * dm))))

    full = lambda shape: pl.BlockSpec(shape, lambda: (0,) * len(shape))

    logits_t = pl.pallas_call(
        _logits_body,
        out_shape=jax.ShapeDtypeStruct((8, T), jnp.float32),
        in_specs=[full((T, S)), full((S, NE)), full((NE, 1))],
        out_specs=full((8, T)),
    )(xf, gate_W, gate_b[:, None])

    tpw = T // (NC * NS)
    gates_w = pl.kernel(
        functools.partial(_gate_sc_body, tpw, T),
        out_type=jax.ShapeDtypeStruct((8 * T,), jnp.float32),
        mesh=plsc.VectorSubcoreMesh(core_axis_name="c", subcore_axis_name="s",
                                    num_cores=NC, num_subcores=NS),
        scratch_types=[pltpu.VMEM((8 * T,), jnp.float32),
                       pltpu.VMEM((8 * tpw,), jnp.float32)],
    )(logits_t.reshape(8 * T))
    gates_t = gates_w.reshape(NC * NS, 8, tpw).transpose(1, 0, 2)

    grid = (T // TILE + 1,)
    shift = lambda i: jnp.maximum(i - 1, 0)
    full_g = lambda shape: pl.BlockSpec(shape,
                                        lambda i: (0,) * len(shape))
    out = pl.pallas_call(
        functools.partial(_fused_body, dims),
        grid=grid,
        out_shape=jax.ShapeDtypeStruct((T, DM), jnp.float32),
        in_specs=[
            pl.BlockSpec((TILE, S), lambda i: (shift(i), 0)),
            pl.BlockSpec((8, TILE), lambda i: (0, shift(i))),
        ]
        + [full_g(w.shape) for w in wf_args]
        + [full_g(w.shape) for w in ff1_W]
        + [full_g((NE, DM)), full_g((NE, DM))],
        out_specs=pl.BlockSpec((TILE, DM), lambda i: (shift(i), 0)),
        scratch_shapes=[pltpu.VMEM((SEQ + 1, NE * DM), jnp.bfloat16),
                        pltpu.VMEM((SEQ + 1, DM), jnp.bfloat16),
                        pltpu.VMEM((DM, DM), jnp.bfloat16)],
    )(xf, gates_t.reshape(8, T), *wf_args, *ff1_W, bf_tiled, b1_stack)

    return out.reshape(B, V, DM)
